# Initial kernel scaffold; baseline (speedup 1.0000x reference)
import functools

import jax
import jax.numpy as jnp
from jax import lax
from jax.experimental import pallas as pl
from jax.experimental.pallas import tpu as pltpu
from jax.experimental.pallas import tpu_sc as plsc

VOCAB = 1002
N_D = 64
BATCH = 4096
HIST = 200

NC = 2
NS = 16
NW = NC * NS

B = BATCH * HIST
BPW = B // NW
GROUP = 1024
NG = BPW // GROUP
SUB = 128
NSUB = GROUP // SUB


def _emb_body(idx_hbm, table_hbm, out_hbm, idx_v, rows_v, gsem):
    wid = lax.axis_index("s") * NC + lax.axis_index("c")
    base = wid * BPW

    def group_step(g, carry):
        row0 = base + g * GROUP
        pltpu.sync_copy(idx_hbm.at[pl.ds(row0, GROUP)], idx_v)
        copies = []
        for j in range(NSUB):
            cp = pltpu.async_copy(
                table_hbm.at[idx_v.at[pl.ds(j * SUB, SUB)]],
                rows_v.at[pl.ds(j * SUB, SUB)],
                gsem,
            )
            copies.append(cp)
        for cp in copies:
            cp.wait()
        pltpu.sync_copy(rows_v, out_hbm.at[pl.ds(row0, GROUP)])
        return carry

    lax.fori_loop(0, NG, group_step, 0)


@jax.jit
def _embedding_sc(idx_flat, table):
    mesh = plsc.VectorSubcoreMesh(
        core_axis_name="c", subcore_axis_name="s",
        num_cores=NC, num_subcores=NS,
    )
    f = functools.partial(
        pl.kernel,
        out_type=jax.ShapeDtypeStruct((B, N_D), jnp.float32),
        mesh=mesh,
        scratch_types=[
            pltpu.VMEM((GROUP,), jnp.int32),
            pltpu.VMEM((GROUP, N_D), jnp.float32),
            pltpu.SemaphoreType.DMA,
        ],
    )(_emb_body)
    return f(idx_flat, table)


def kernel(input, table):
    idx_flat = input.reshape(-1).astype(jnp.int32)
    out = _embedding_sc(idx_flat, table)
    return out.reshape(BATCH, HIST, N_D)


# speedup vs baseline: 3.5873x; 3.5873x over previous
import functools

import jax
import jax.numpy as jnp
from jax import lax
from jax.experimental import pallas as pl
from jax.experimental.pallas import tpu as pltpu
from jax.experimental.pallas import tpu_sc as plsc

VOCAB = 1002
N_D = 64
BATCH = 4096
HIST = 200

NC = 2
NS = 16
NW = NC * NS

B = BATCH * HIST
BPW = B // NW
GROUP = 1024
NG = BPW // GROUP
SUB = 128
NSUB = GROUP // SUB


def _emb_body(idx_hbm, table_hbm, out_hbm, idx_v, rows_v, gsem):
    wid = lax.axis_index("s") * NC + lax.axis_index("c")
    base = wid * BPW

    def group_step(g, carry):
        row0 = base + g * GROUP
        pltpu.sync_copy(idx_hbm.at[pl.ds(row0, GROUP)], idx_v)
        copies = []
        for j in range(NSUB):
            cp = pltpu.async_copy(
                table_hbm.at[idx_v.at[pl.ds(j * SUB, SUB)]],
                rows_v.at[pl.ds(j * SUB, SUB)],
                gsem,
            )
            copies.append(cp)
        for cp in copies:
            cp.wait()
        pltpu.sync_copy(rows_v, out_hbm.at[pl.ds(row0, GROUP)])
        return carry

    lax.fori_loop(0, NG, group_step, 0)


@jax.jit
def _embedding_sc(idx_flat, table):
    mesh = plsc.VectorSubcoreMesh(
        core_axis_name="c", subcore_axis_name="s",
        num_cores=NC, num_subcores=NS,
    )
    f = functools.partial(
        pl.kernel,
        out_type=jax.ShapeDtypeStruct((B, N_D), jnp.float32),
        mesh=mesh,
        scratch_types=[
            pltpu.VMEM((GROUP,), jnp.int32),
            pltpu.VMEM((GROUP, N_D), jnp.float32),
            pltpu.SemaphoreType.DMA,
        ],
        compiler_params=pltpu.CompilerParams(use_tc_tiling_on_sc=False),
    )(_emb_body)
    return f(idx_flat, table)


def kernel(input, table):
    idx_flat = input.reshape(-1).astype(jnp.int32)
    out = _embedding_sc(idx_flat, table)
    return out.reshape(BATCH, HIST, N_D)
